# Initial kernel scaffold; baseline (speedup 1.0000x reference)
import functools

import jax
import jax.numpy as jnp
from jax import lax
from jax.experimental import pallas as pl
from jax.experimental.pallas import tpu as pltpu
from jax.experimental.pallas import tpu_sc as plsc

N = 100000
E = 6400000
STEPS = 5

NTILES = 32
NSUB = 16
NP = 100352
EP = 6422528
EPT = EP // NTILES
SUB = 128
C = 2048
ROWS_PER_CHUNK = C // SUB
NCHUNK = EPT // C
NSLICE = NP // NSUB
LN2 = 0.6931471805599453


def _sc_step(row2, col2, w1, x1):
    mesh = plsc.VectorSubcoreMesh(core_axis_name="c", subcore_axis_name="s")

    @functools.partial(
        pl.kernel,
        mesh=mesh,
        out_type=jax.ShapeDtypeStruct((2, NP), jnp.float32),
        scratch_types=[
            pltpu.VMEM_SHARED((NP,), jnp.float32),
            pltpu.VMEM_SHARED((NP,), jnp.float32),
            pltpu.VMEM((ROWS_PER_CHUNK, SUB), jnp.int32),
            pltpu.VMEM((ROWS_PER_CHUNK, SUB), jnp.int32),
            pltpu.VMEM((C,), jnp.float32),
            pltpu.VMEM((C,), jnp.float32),
            pltpu.VMEM((C,), jnp.float32),
            pltpu.VMEM((NSLICE,), jnp.float32),
            pltpu.SemaphoreType.DMA,
        ],
    )
    def k(row_h, col_h, w_h, x_h, agg_out, x_sp, agg_sp,
          row_b, col_b, w_b, xg_b, msg_b, zb, gsem):
        c = lax.axis_index("c")
        s = lax.axis_index("s")
        wid = c * NSUB + s
        nbase = s * NSLICE

        def zfill(i, _):
            zb[pl.ds(i * 16, 16)] = jnp.zeros((16,), jnp.float32)
            return 0
        lax.fori_loop(0, NSLICE // 16, zfill, 0)
        pltpu.sync_copy(x_h.at[pl.ds(nbase, NSLICE)],
                        x_sp.at[pl.ds(nbase, NSLICE)])
        pltpu.sync_copy(zb, agg_sp.at[pl.ds(nbase, NSLICE)])
        plsc.subcore_barrier()

        row_base = wid * (EPT // SUB)

        def chunk(i, _):
            rbase = row_base + i * ROWS_PER_CHUNK
            pltpu.sync_copy(row_h.at[pl.ds(rbase, ROWS_PER_CHUNK)], row_b)
            pltpu.sync_copy(col_h.at[pl.ds(rbase, ROWS_PER_CHUNK)], col_b)
            pltpu.sync_copy(w_h.at[pl.ds(rbase * SUB, C)], w_b)

            hs = [pltpu.async_copy(x_sp.at[row_b.at[j]],
                                   xg_b.at[pl.ds(j * SUB, SUB)], gsem)
                  for j in range(ROWS_PER_CHUNK)]
            for h in hs:
                h.wait()

            def compute(kk, _):
                wv = w_b[pl.ds(kk * 16, 16)]
                xv = xg_b[pl.ds(kk * 16, 16)]
                t = (1.0 - wv * xv) + 1e-15
                bits = plsc.bitcast(t, jnp.int32)
                e = lax.shift_right_arithmetic(bits, 23) - 127
                mb = lax.bitwise_or(lax.bitwise_and(bits, 0x007FFFFF),
                                    0x3F800000)
                m = plsc.bitcast(mb, jnp.float32)
                big = m > 1.4142135
                m2 = jnp.where(big, m * 0.5, m)
                ef = e.astype(jnp.float32) + jnp.where(big, 1.0, 0.0)
                u = (m2 - 1.0) / (m2 + 1.0)
                z = u * u
                lnm = u * (2.0 + z * (0.66666667 + z * (0.4 + z * 0.28571429)))
                msg_b[pl.ds(kk * 16, 16)] = ef * LN2 + lnm
                return 0
            lax.fori_loop(0, C // 16, compute, 0)

            for j in range(ROWS_PER_CHUNK):
                pltpu.sync_copy(msg_b.at[pl.ds(j * SUB, SUB)],
                                agg_sp.at[col_b.at[j]], add=True)
            return 0

        lax.fori_loop(0, NCHUNK, chunk, 0)
        plsc.subcore_barrier()
        pltpu.sync_copy(agg_sp.at[pl.ds(nbase, NSLICE)],
                        agg_out.at[c].at[pl.ds(nbase, NSLICE)])

    return k(row2, col2, w1, x1)


def _tc_update(agg2, s, x, r):
    def body(a_ref, s_ref, x_ref, r_ref, so_ref, xo_ref, ro_ref):
        q = jnp.exp(a_ref[0] + a_ref[1])
        sv = s_ref[...]
        so_ref[...] = sv * q
        xo_ref[...] = sv * (1.0 - q)
        ro_ref[...] = r_ref[...] + x_ref[...]

    shp = jax.ShapeDtypeStruct((NP // 128, 128), jnp.float32)
    return pl.pallas_call(
        body,
        out_shape=(shp, shp, shp),
    )(agg2.reshape(2, NP // 128, 128), s, x, r)


def kernel(edge_index, edge_weight, x0):
    row = edge_index[0]
    col = edge_index[1]
    w = edge_weight[:, 0]
    x = x0[:, 0]

    pad_e = EP - E
    rowp = jnp.concatenate([row, jnp.zeros((pad_e,), jnp.int32)])
    colp = jnp.concatenate([col, jnp.full((pad_e,), N, jnp.int32)])
    wp = jnp.concatenate([w, jnp.zeros((pad_e,), jnp.float32)])
    row2 = rowp.reshape(EP // SUB, SUB)
    col2 = colp.reshape(EP // SUB, SUB)

    xp = jnp.pad(x, (0, NP - N)).reshape(NP // 128, 128)
    sp = jnp.pad(1.0 - x, (0, NP - N)).reshape(NP // 128, 128)
    rp = jnp.zeros((NP // 128, 128), jnp.float32)

    for _ in range(STEPS):
        agg2 = _sc_step(row2, col2, wp, xp.reshape(NP))
        sp, xp, rp = _tc_update(agg2, sp, xp, rp)

    s_out = sp.reshape(NP)[:N, None]
    x_out = xp.reshape(NP)[:N, None]
    r_out = rp.reshape(NP)[:N, None]
    return (s_out, x_out, r_out)


# speedup vs baseline: 124.9681x; 124.9681x over previous
import functools

import jax
import jax.numpy as jnp
from jax import lax
from jax.experimental import pallas as pl
from jax.experimental.pallas import tpu as pltpu
from jax.experimental.pallas import tpu_sc as plsc

N = 100000
E = 6400000
STEPS = 5

NTILES = 32
NSUB = 16
NP = 100352
EP = 6422528
EPT = EP // NTILES
SUB = 128
C = 2048
ROWS_PER_CHUNK = C // SUB
NCHUNK = EPT // C
NSLICE = NP // NSUB
LN2 = 0.6931471805599453


def _sc_step(row2, col2, w1, x1):
    mesh = plsc.VectorSubcoreMesh(core_axis_name="c", subcore_axis_name="s")

    @functools.partial(
        pl.kernel,
        mesh=mesh,
        out_type=jax.ShapeDtypeStruct((2, NP), jnp.float32),
        scratch_types=[
            pltpu.VMEM_SHARED((NP,), jnp.float32),
            pltpu.VMEM_SHARED((NP,), jnp.float32),
            pltpu.VMEM((ROWS_PER_CHUNK, SUB), jnp.int32),
            pltpu.VMEM((ROWS_PER_CHUNK, SUB), jnp.int32),
            pltpu.VMEM((C,), jnp.float32),
            pltpu.VMEM((C,), jnp.float32),
            pltpu.VMEM((C,), jnp.float32),
            pltpu.VMEM((NSLICE,), jnp.float32),
            pltpu.SemaphoreType.DMA,
        ],
    )
    def k(row_h, col_h, w_h, x_h, agg_out, x_sp, agg_sp,
          row_b, col_b, w_b, xg_b, msg_b, zb, gsem):
        c = lax.axis_index("c")
        s = lax.axis_index("s")
        wid = c * NSUB + s
        nbase = s * NSLICE

        def zfill(i, _):
            zb[pl.ds(i * 16, 16)] = jnp.zeros((16,), jnp.float32)
            return 0
        lax.fori_loop(0, NSLICE // 16, zfill, 0)
        pltpu.sync_copy(x_h.at[pl.ds(nbase, NSLICE)],
                        x_sp.at[pl.ds(nbase, NSLICE)])
        pltpu.sync_copy(zb, agg_sp.at[pl.ds(nbase, NSLICE)])
        plsc.subcore_barrier()

        row_base = wid * (EPT // SUB)

        def chunk(i, _):
            rbase = row_base + i * ROWS_PER_CHUNK
            pltpu.sync_copy(row_h.at[pl.ds(rbase, ROWS_PER_CHUNK)], row_b)
            pltpu.sync_copy(col_h.at[pl.ds(rbase, ROWS_PER_CHUNK)], col_b)
            pltpu.sync_copy(w_h.at[pl.ds(rbase * SUB, C)], w_b)

            hs = [pltpu.async_copy(x_sp.at[row_b.at[j]],
                                   xg_b.at[pl.ds(j * SUB, SUB)], gsem)
                  for j in range(ROWS_PER_CHUNK)]
            for h in hs:
                h.wait()

            def compute(kk, _):
                wv = w_b[pl.ds(kk * 16, 16)]
                xv = xg_b[pl.ds(kk * 16, 16)]
                t = (1.0 - wv * xv) + 1e-15
                bits = lax.bitcast_convert_type(t, jnp.int32)
                e = lax.shift_right_arithmetic(bits, 23) - 127
                mb = lax.bitwise_or(lax.bitwise_and(bits, 0x007FFFFF),
                                    0x3F800000)
                m = lax.bitcast_convert_type(mb, jnp.float32)
                big = m > 1.4142135
                m2 = jnp.where(big, m * 0.5, m)
                ef = e.astype(jnp.float32) + jnp.where(big, 1.0, 0.0)
                u = (m2 - 1.0) / (m2 + 1.0)
                z = u * u
                lnm = u * (2.0 + z * (0.66666667 + z * (0.4 + z * 0.28571429)))
                msg_b[pl.ds(kk * 16, 16)] = ef * LN2 + lnm
                return 0
            lax.fori_loop(0, C // 16, compute, 0)

            for j in range(ROWS_PER_CHUNK):
                pltpu.sync_copy(msg_b.at[pl.ds(j * SUB, SUB)],
                                agg_sp.at[col_b.at[j]], add=True)
            return 0

        lax.fori_loop(0, NCHUNK, chunk, 0)
        plsc.subcore_barrier()
        pltpu.sync_copy(agg_sp.at[pl.ds(nbase, NSLICE)],
                        agg_out.at[c].at[pl.ds(nbase, NSLICE)])

    return k(row2, col2, w1, x1)


def _tc_update(agg2, s, x, r):
    def body(a_ref, s_ref, x_ref, r_ref, so_ref, xo_ref, ro_ref):
        q = jnp.exp(a_ref[0] + a_ref[1])
        sv = s_ref[...]
        so_ref[...] = sv * q
        xo_ref[...] = sv * (1.0 - q)
        ro_ref[...] = r_ref[...] + x_ref[...]

    shp = jax.ShapeDtypeStruct((NP // 128, 128), jnp.float32)
    return pl.pallas_call(
        body,
        out_shape=(shp, shp, shp),
    )(agg2.reshape(2, NP // 128, 128), s, x, r)


def kernel(edge_index, edge_weight, x0):
    row = edge_index[0]
    col = edge_index[1]
    w = edge_weight[:, 0]
    x = x0[:, 0]

    pad_e = EP - E
    rowp = jnp.concatenate([row, jnp.zeros((pad_e,), jnp.int32)])
    colp = jnp.concatenate([col, jnp.full((pad_e,), N, jnp.int32)])
    wp = jnp.concatenate([w, jnp.zeros((pad_e,), jnp.float32)])
    row2 = rowp.reshape(EP // SUB, SUB)
    col2 = colp.reshape(EP // SUB, SUB)

    xp = jnp.pad(x, (0, NP - N)).reshape(NP // 128, 128)
    sp = jnp.pad(1.0 - x, (0, NP - N)).reshape(NP // 128, 128)
    rp = jnp.zeros((NP // 128, 128), jnp.float32)

    for _ in range(STEPS):
        agg2 = _sc_step(row2, col2, wp, xp.reshape(NP))
        sp, xp, rp = _tc_update(agg2, sp, xp, rp)

    s_out = sp.reshape(NP)[:N, None]
    x_out = xp.reshape(NP)[:N, None]
    r_out = rp.reshape(NP)[:N, None]
    return (s_out, x_out, r_out)


# speedup vs baseline: 134.9501x; 1.0799x over previous
import functools

import jax
import jax.numpy as jnp
from jax import lax
from jax.experimental import pallas as pl
from jax.experimental.pallas import tpu as pltpu
from jax.experimental.pallas import tpu_sc as plsc

N = 100000
E = 6400000
STEPS = 5

NTILES = 32
NSUB = 16
NP = 100352
SUB = 128
C = 1024
ROWS_PER_CHUNK = C // SUB
NCHUNK = 200
EPT = C * NCHUNK
EP = EPT * NTILES
NSLICE = NP // NSUB
LN2 = 0.6931471805599453


def _sc_step(row1, col2, w1, x1):
    mesh = plsc.VectorSubcoreMesh(core_axis_name="c", subcore_axis_name="s")

    @functools.partial(
        pl.kernel,
        mesh=mesh,
        compiler_params=pltpu.CompilerParams(needs_layout_passes=False),
        out_type=jax.ShapeDtypeStruct((2, NP), jnp.float32),
        scratch_types=[
            pltpu.VMEM_SHARED((NP,), jnp.float32),
            pltpu.VMEM((NP,), jnp.float32),
            pltpu.VMEM((2, C), jnp.int32),
            pltpu.VMEM((4, ROWS_PER_CHUNK, SUB), jnp.int32),
            pltpu.VMEM((2, C), jnp.float32),
            pltpu.VMEM((4, C), jnp.float32),
            pltpu.VMEM((NSLICE // 2,), jnp.float32),
            pltpu.SemaphoreType.DMA,
            pltpu.SemaphoreType.DMA,
            pltpu.SemaphoreType.DMA,
        ],
    )
    def k(row_h, col_h, w_h, x_h, agg_out, agg_sp, x_tl,
          row_b, col_b, w_b, msg_b, zb, rwsem, csem, ssem):
        c = lax.axis_index("c")
        s = lax.axis_index("s")
        wid = c * NSUB + s
        nbase = s * NSLICE
        ebase = wid * EPT
        rowbase = wid * (EPT // SUB)

        def zfill(i, _):
            zb[pl.ds(i * 16, 16)] = jnp.zeros((16,), jnp.float32)
            return 0
        lax.fori_loop(0, NSLICE // 32, zfill, 0)
        pltpu.sync_copy(x_h, x_tl)
        pltpu.sync_copy(zb, agg_sp.at[pl.ds(nbase, NSLICE // 2)])
        pltpu.sync_copy(zb, agg_sp.at[pl.ds(nbase + NSLICE // 2, NSLICE // 2)])
        plsc.subcore_barrier()

        def start_rw(ic, pr):
            eb = ebase + ic * C
            pltpu.async_copy(row_h.at[pl.ds(eb, C)], row_b.at[pr], rwsem)
            pltpu.async_copy(w_h.at[pl.ds(eb, C)], w_b.at[pr], rwsem)

        def wait_rw(pr):
            pltpu.make_async_copy(row_h.at[pl.ds(0, C)],
                                  row_b.at[pr], rwsem).wait()
            pltpu.make_async_copy(w_h.at[pl.ds(0, C)],
                                  w_b.at[pr], rwsem).wait()

        def start_col(ic, pc):
            rb = rowbase + ic * ROWS_PER_CHUNK
            pltpu.async_copy(col_h.at[pl.ds(rb, ROWS_PER_CHUNK)],
                             col_b.at[pc], csem)

        def wait_col(pc):
            pltpu.make_async_copy(col_h.at[pl.ds(0, ROWS_PER_CHUNK)],
                                  col_b.at[pc], csem).wait()

        def fire_scatter(pc, pm):
            for j in range(ROWS_PER_CHUNK):
                pltpu.async_copy(msg_b.at[pm, pl.ds(j * SUB, SUB)],
                                 agg_sp.at[col_b.at[pc, j]], ssem, add=True)

        def drain_scatter(pc, pm):
            for j in range(ROWS_PER_CHUNK):
                pltpu.make_async_copy(msg_b.at[pm, pl.ds(j * SUB, SUB)],
                                      agg_sp.at[col_b.at[pc, j]], ssem).wait()

        def compute(pr, pm):
            def body(kk, _):
                sl = pl.ds(kk * 16, 16)
                rv = row_b[pr, sl]
                xv = plsc.load_gather(x_tl, [rv])
                wv = w_b[pr, sl]
                t = (1.0 - wv * xv) + 1e-15
                bits = lax.bitcast_convert_type(t, jnp.int32)
                e = lax.shift_right_arithmetic(bits, 23) - 127
                mb = lax.bitwise_or(lax.bitwise_and(bits, 0x007FFFFF),
                                    0x3F800000)
                m = lax.bitcast_convert_type(mb, jnp.float32)
                big = m > 1.4142135
                m2 = jnp.where(big, m * 0.5, m)
                ef = e.astype(jnp.float32) + jnp.where(big, 1.0, 0.0)
                u = (m2 - 1.0) / (m2 + 1.0)
                z = u * u
                lnm = u * (2.0 + z * (0.66666667 + z * (0.4 + z * 0.28571429)))
                msg_b[pm, sl] = ef * LN2 + lnm
                return 0
            lax.fori_loop(0, C // 16, body, 0)

        def chunk_work(i_dyn, kph, do_drain):
            pr, pc, pm = kph % 2, kph % 4, kph % 4
            wait_rw(pr)
            wait_col(pc)
            if do_drain:
                drain_scatter((kph + 2) % 4, (kph + 2) % 4)
            i1 = jnp.minimum(i_dyn + 1, NCHUNK - 1)
            start_rw(i1, (kph + 1) % 2)
            i2 = jnp.minimum(i_dyn + 2, NCHUNK - 1)
            start_col(i2, (kph + 2) % 4)
            compute(pr, pm)
            fire_scatter(pc, pm)

        start_rw(0, 0)
        start_col(0, 0)
        start_col(1, 1)
        chunk_work(0, 0, False)
        chunk_work(1, 1, False)
        chunk_work(2, 2, True)
        chunk_work(3, 3, True)

        def group(j, _):
            i4 = j * 4
            for kph in range(4):
                chunk_work(i4 + kph, kph, True)
            return 0
        lax.fori_loop(1, NCHUNK // 4, group, 0)

        drain_scatter(2, 2)
        drain_scatter(3, 3)
        wait_rw(0)
        wait_col(0)
        wait_col(1)

        plsc.subcore_barrier()
        pltpu.sync_copy(agg_sp.at[pl.ds(nbase, NSLICE)],
                        agg_out.at[c].at[pl.ds(nbase, NSLICE)])

    return k(row1, col2, w1, x1)


def _tc_update(agg2, s, x, r):
    def body(a_ref, s_ref, x_ref, r_ref, so_ref, xo_ref, ro_ref):
        q = jnp.exp(a_ref[0] + a_ref[1])
        sv = s_ref[...]
        so_ref[...] = sv * q
        xo_ref[...] = sv * (1.0 - q)
        ro_ref[...] = r_ref[...] + x_ref[...]

    shp = jax.ShapeDtypeStruct((NP // 128, 128), jnp.float32)
    return pl.pallas_call(
        body,
        out_shape=(shp, shp, shp),
    )(agg2.reshape(2, NP // 128, 128), s, x, r)


def kernel(edge_index, edge_weight, x0):
    row = edge_index[0]
    col = edge_index[1]
    w = edge_weight[:, 0]
    x = x0[:, 0]

    pad_e = EP - E
    rowp = jnp.concatenate([row, jnp.zeros((pad_e,), jnp.int32)])
    colp = jnp.concatenate([col, jnp.full((pad_e,), N, jnp.int32)])
    wp = jnp.concatenate([w, jnp.zeros((pad_e,), jnp.float32)])
    col2 = colp.reshape(EP // SUB, SUB)

    xp = jnp.pad(x, (0, NP - N)).reshape(NP // 128, 128)
    sp = jnp.pad(1.0 - x, (0, NP - N)).reshape(NP // 128, 128)
    rp = jnp.zeros((NP // 128, 128), jnp.float32)

    for _ in range(STEPS):
        agg2 = _sc_step(rowp, col2, wp, xp.reshape(NP))
        sp, xp, rp = _tc_update(agg2, sp, xp, rp)

    s_out = sp.reshape(NP)[:N, None]
    x_out = xp.reshape(NP)[:N, None]
    r_out = rp.reshape(NP)[:N, None]
    return (s_out, x_out, r_out)


# speedup vs baseline: 138.9811x; 1.0299x over previous
import functools

import jax
import jax.numpy as jnp
from jax import lax
from jax.experimental import pallas as pl
from jax.experimental.pallas import tpu as pltpu
from jax.experimental.pallas import tpu_sc as plsc

N = 100000
E = 6400000
STEPS = 5

NTILES = 32
NSUB = 16
NP = 100352
SUB = 128
C = 1024
ROWS_PER_CHUNK = C // SUB
NCHUNK = 200
EPT = C * NCHUNK
EP = EPT * NTILES
NSLICE = NP // NSUB
LN2 = 0.6931471805599453


def _sc_step(row1, col2, w1, x1):
    mesh = plsc.VectorSubcoreMesh(core_axis_name="c", subcore_axis_name="s")

    @functools.partial(
        pl.kernel,
        mesh=mesh,
        compiler_params=pltpu.CompilerParams(needs_layout_passes=False),
        out_type=jax.ShapeDtypeStruct((2, NP), jnp.float32),
        scratch_types=[
            pltpu.VMEM_SHARED((NP,), jnp.float32),
            pltpu.VMEM((NP,), jnp.float32),
            pltpu.VMEM((2, C), jnp.int32),
            pltpu.VMEM((4, ROWS_PER_CHUNK, SUB), jnp.int32),
            pltpu.VMEM((2, C), jnp.float32),
            pltpu.VMEM((4, C), jnp.float32),
            pltpu.VMEM((NSLICE // 2,), jnp.float32),
            pltpu.SemaphoreType.DMA,
            pltpu.SemaphoreType.DMA,
            pltpu.SemaphoreType.DMA,
        ],
    )
    def k(row_h, col_h, w_h, x_h, agg_out, agg_sp, x_tl,
          row_b, col_b, w_b, msg_b, zb, rwsem, csem, ssem):
        c = lax.axis_index("c")
        s = lax.axis_index("s")
        wid = c * NSUB + s
        nbase = s * NSLICE
        ebase = wid * EPT
        rowbase = wid * (EPT // SUB)

        def zfill(i, _):
            zb[pl.ds(i * 16, 16)] = jnp.zeros((16,), jnp.float32)
            return 0
        lax.fori_loop(0, NSLICE // 32, zfill, 0)
        pltpu.sync_copy(x_h, x_tl)
        pltpu.sync_copy(zb, agg_sp.at[pl.ds(nbase, NSLICE // 2)])
        pltpu.sync_copy(zb, agg_sp.at[pl.ds(nbase + NSLICE // 2, NSLICE // 2)])
        plsc.subcore_barrier()

        def start_rw(ic, pr):
            eb = ebase + ic * C
            pltpu.async_copy(row_h.at[pl.ds(eb, C)], row_b.at[pr], rwsem)
            pltpu.async_copy(w_h.at[pl.ds(eb, C)], w_b.at[pr], rwsem)

        def wait_rw(pr):
            pltpu.make_async_copy(row_h.at[pl.ds(0, C)],
                                  row_b.at[pr], rwsem).wait()
            pltpu.make_async_copy(w_h.at[pl.ds(0, C)],
                                  w_b.at[pr], rwsem).wait()

        def start_col(ic, pc):
            rb = rowbase + ic * ROWS_PER_CHUNK
            pltpu.async_copy(col_h.at[pl.ds(rb, ROWS_PER_CHUNK)],
                             col_b.at[pc], csem)

        def wait_col(pc):
            pltpu.make_async_copy(col_h.at[pl.ds(0, ROWS_PER_CHUNK)],
                                  col_b.at[pc], csem).wait()

        def fire_scatter(pc, pm):
            for j in range(ROWS_PER_CHUNK):
                pltpu.async_copy(msg_b.at[pm, pl.ds(j * SUB, SUB)],
                                 agg_sp.at[col_b.at[pc, j]], ssem, add=True)

        def drain_scatter(pc, pm):
            for j in range(ROWS_PER_CHUNK):
                pltpu.make_async_copy(msg_b.at[pm, pl.ds(j * SUB, SUB)],
                                      agg_sp.at[col_b.at[pc, j]], ssem).wait()

        def compute(pr, pm):
            P = (1.0000009643975858, -0.5000114503774549, 0.3331467380854648,
                 -0.2490828918472631, 0.20491759650034064,
                 -0.1868075142713013, 0.11931054435719697)

            def one(kk, uu):
                sl = pl.ds(kk * 64 + uu * 16, 16)
                rv = row_b[pr, sl]
                xv = plsc.load_gather(x_tl, [rv])
                wv = w_b[pr, sl]
                t = (1.0 - wv * xv) + 1e-15
                bits = lax.bitcast_convert_type(t, jnp.int32)
                e = lax.shift_right_arithmetic(bits, 23) - 127
                mb = lax.bitwise_or(lax.bitwise_and(bits, 0x007FFFFF),
                                    0x3F800000)
                m = lax.bitcast_convert_type(mb, jnp.float32)
                big = m > 1.4142135
                d = jnp.where(big, m * 0.5, m) - 1.0
                ef = e.astype(jnp.float32) + jnp.where(big, 1.0, 0.0)
                g = P[6]
                for cc in (P[5], P[4], P[3], P[2], P[1], P[0]):
                    g = g * d + cc
                msg_b[pm, sl] = ef * LN2 + d * g

            def body(kk, _):
                for uu in range(4):
                    one(kk, uu)
                return 0
            lax.fori_loop(0, C // 64, body, 0)

        def chunk_work(i_dyn, kph, do_drain):
            pr, pc, pm = kph % 2, kph % 4, kph % 4
            wait_rw(pr)
            wait_col(pc)
            if do_drain:
                drain_scatter((kph + 2) % 4, (kph + 2) % 4)
            i1 = jnp.minimum(i_dyn + 1, NCHUNK - 1)
            start_rw(i1, (kph + 1) % 2)
            i2 = jnp.minimum(i_dyn + 2, NCHUNK - 1)
            start_col(i2, (kph + 2) % 4)
            compute(pr, pm)
            fire_scatter(pc, pm)

        start_rw(0, 0)
        start_col(0, 0)
        start_col(1, 1)
        chunk_work(0, 0, False)
        chunk_work(1, 1, False)
        chunk_work(2, 2, True)
        chunk_work(3, 3, True)

        def group(j, _):
            i4 = j * 4
            for kph in range(4):
                chunk_work(i4 + kph, kph, True)
            return 0
        lax.fori_loop(1, NCHUNK // 4, group, 0)

        drain_scatter(2, 2)
        drain_scatter(3, 3)
        wait_rw(0)
        wait_col(0)
        wait_col(1)

        plsc.subcore_barrier()
        pltpu.sync_copy(agg_sp.at[pl.ds(nbase, NSLICE)],
                        agg_out.at[c].at[pl.ds(nbase, NSLICE)])

    return k(row1, col2, w1, x1)


def _tc_update(agg2, s, x, r):
    def body(a_ref, s_ref, x_ref, r_ref, so_ref, xo_ref, ro_ref):
        q = jnp.exp(a_ref[0] + a_ref[1])
        sv = s_ref[...]
        so_ref[...] = sv * q
        xo_ref[...] = sv * (1.0 - q)
        ro_ref[...] = r_ref[...] + x_ref[...]

    shp = jax.ShapeDtypeStruct((NP // 128, 128), jnp.float32)
    return pl.pallas_call(
        body,
        out_shape=(shp, shp, shp),
    )(agg2.reshape(2, NP // 128, 128), s, x, r)


def kernel(edge_index, edge_weight, x0):
    row = edge_index[0]
    col = edge_index[1]
    w = edge_weight[:, 0]
    x = x0[:, 0]

    pad_e = EP - E
    rowp = jnp.concatenate([row, jnp.zeros((pad_e,), jnp.int32)])
    colp = jnp.concatenate([col, jnp.full((pad_e,), N, jnp.int32)])
    wp = jnp.concatenate([w, jnp.zeros((pad_e,), jnp.float32)])
    col2 = colp.reshape(EP // SUB, SUB)

    xp = jnp.pad(x, (0, NP - N)).reshape(NP // 128, 128)
    sp = jnp.pad(1.0 - x, (0, NP - N)).reshape(NP // 128, 128)
    rp = jnp.zeros((NP // 128, 128), jnp.float32)

    for _ in range(STEPS):
        agg2 = _sc_step(rowp, col2, wp, xp.reshape(NP))
        sp, xp, rp = _tc_update(agg2, sp, xp, rp)

    s_out = sp.reshape(NP)[:N, None]
    x_out = xp.reshape(NP)[:N, None]
    r_out = rp.reshape(NP)[:N, None]
    return (s_out, x_out, r_out)
